# baseline (speedup 1.0000x reference)
import functools

import jax
import jax.numpy as jnp
from jax import lax
from jax.experimental import pallas as pl
from jax.experimental.pallas import tpu as pltpu
from jax.experimental.pallas import tpu_sc as plsc

_NUM_EMBEDDINGS = 1024
_EMBEDDING_DIM = 64
_BLOCK = 2048
_N_TOKENS = 8192
_NC, _NS = 2, 16
_NW = _NC * _NS
_BPW = _N_TOKENS // _NW


def _vq_kernel(n_tokens, grid, x_ref, emb_ref, x2_ref, e2_ref, idx_ref,
               loss_ref, loss_scr):
    i = pl.program_id(0)
    x = x_ref[...]
    emb = emb_ref[...]
    xt = x.T
    mm2_t = jnp.dot(emb + emb, xt,
                    preferred_element_type=jnp.float32)
    scores_t = (x2_ref[...] - mm2_t) + e2_ref[...]
    minval = jnp.min(scores_t, axis=0)
    iota_c = jax.lax.broadcasted_iota(jnp.int32, (_NUM_EMBEDDINGS, _BLOCK), 0)
    masked = jnp.where(scores_t == minval[None, :], iota_c,
                       _NUM_EMBEDDINGS)
    idx = jnp.min(masked, axis=0).astype(jnp.int32)
    idx_ref[0, 0, :] = idx

    part_loss = jnp.sum(minval)

    @pl.when(i == 0)
    def _init():
        loss_scr[0, 0] = 0.0

    loss_scr[0, 0] += part_loss

    @pl.when(i == grid - 1)
    def _finalize():
        loss_ref[0, 0] = loss_scr[0, 0] / (n_tokens * _EMBEDDING_DIM)


_sc_mesh = plsc.VectorSubcoreMesh(core_axis_name="c", subcore_axis_name="s")


@functools.partial(
    pl.kernel, mesh=_sc_mesh,
    out_type=[
        jax.ShapeDtypeStruct((_N_TOKENS, 2 * _EMBEDDING_DIM), jnp.float32),
        jax.ShapeDtypeStruct((_NC, _NUM_EMBEDDINGS), jnp.float32),
    ],
    scratch_types=[
        pltpu.VMEM((_BPW,), jnp.int32),
        pltpu.VMEM((_BPW, 2 * _EMBEDDING_DIM), jnp.float32),
        pltpu.VMEM((_BPW,), jnp.float32),
        pltpu.VMEM((_NUM_EMBEDDINGS,), jnp.float32),
        pltpu.VMEM_SHARED((_NUM_EMBEDDINGS,), jnp.float32),
        pltpu.SemaphoreType.DMA,
    ],
)
def _sc_gather_hist(table_hbm, idx_hbm, out_hbm, counts_hbm,
                    idx_v, rows_v, ones_v, zeros_v, shared_counts, sem):
    c = lax.axis_index("c")
    s = lax.axis_index("s")
    wid = s * _NC + c
    base = wid * _BPW
    pltpu.sync_copy(idx_hbm.at[pl.ds(base, _BPW)], idx_v)
    pltpu.async_copy(table_hbm.at[idx_v], rows_v, sem).wait()
    pltpu.sync_copy(rows_v, out_hbm.at[pl.ds(base, _BPW)])

    @pl.when(s == 0)
    def _init():
        for j in range(_NUM_EMBEDDINGS // 16):
            zeros_v[pl.ds(16 * j, 16)] = jnp.zeros((16,), jnp.float32)
        pltpu.sync_copy(zeros_v, shared_counts)

    for j in range(_BPW // 16):
        ones_v[pl.ds(16 * j, 16)] = jnp.ones((16,), jnp.float32)
    plsc.subcore_barrier()
    pltpu.sync_copy(ones_v, shared_counts.at[idx_v], add=True)
    plsc.subcore_barrier()

    @pl.when(s == 0)
    def _out():
        pltpu.sync_copy(shared_counts, counts_hbm.at[c])


def _perp_kernel(n_tokens, counts_ref, perp_ref):
    counts = counts_ref[0, :] + counts_ref[1, :]
    p = counts * (1.0 / n_tokens)
    perp_ref[0, 0] = jnp.exp(-jnp.sum(p * jnp.log(p + 1e-10)))


def kernel(inputs, embedding):
    input_shape = inputs.shape
    flat = inputs.reshape(-1, _EMBEDDING_DIM)
    n_tokens = flat.shape[0]
    grid = n_tokens // _BLOCK
    x2 = jnp.sum(flat ** 2, axis=1).reshape(1, n_tokens)
    e2 = jnp.sum(embedding ** 2, axis=1).reshape(_NUM_EMBEDDINGS, 1)
    emb_pad = jnp.concatenate(
        [embedding, jnp.zeros((_NUM_EMBEDDINGS, _EMBEDDING_DIM), jnp.float32)],
        axis=1)

    idx3, loss = pl.pallas_call(
        functools.partial(_vq_kernel, n_tokens, grid),
        grid=(grid,),
        in_specs=[
            pl.BlockSpec((_BLOCK, _EMBEDDING_DIM), lambda i: (i, 0)),
            pl.BlockSpec((_NUM_EMBEDDINGS, _EMBEDDING_DIM), lambda i: (0, 0)),
            pl.BlockSpec((1, _BLOCK), lambda i: (0, i)),
            pl.BlockSpec((_NUM_EMBEDDINGS, 1), lambda i: (0, 0)),
        ],
        out_specs=[
            pl.BlockSpec((1, 1, _BLOCK), lambda i: (i, 0, 0)),
            pl.BlockSpec(memory_space=pltpu.SMEM, block_shape=(1, 1),
                         index_map=lambda i: (0, 0)),
        ],
        out_shape=[
            jax.ShapeDtypeStruct((grid, 1, _BLOCK), jnp.int32),
            jax.ShapeDtypeStruct((1, 1), jnp.float32),
        ],
        scratch_shapes=[
            pltpu.SMEM((1, 1), jnp.float32),
        ],
    )(flat, embedding, x2, e2)

    idx_flat = idx3.reshape(n_tokens)
    q_pad, counts2 = _sc_gather_hist(emb_pad, idx_flat)

    perp = pl.pallas_call(
        functools.partial(_perp_kernel, n_tokens),
        in_specs=[pl.BlockSpec((_NC, _NUM_EMBEDDINGS), lambda: (0, 0))],
        out_specs=pl.BlockSpec(memory_space=pltpu.SMEM, block_shape=(1, 1),
                               index_map=lambda: (0, 0)),
        out_shape=jax.ShapeDtypeStruct((1, 1), jnp.float32),
    )(counts2)

    quantized = q_pad[:, :_EMBEDDING_DIM].reshape(input_shape)
    indices = idx3.reshape(input_shape[:-1])
    return (quantized, loss.reshape(()), indices, perp.reshape(()))


# speedup vs baseline: 1.5386x; 1.5386x over previous
import functools

import jax
import jax.numpy as jnp
from jax.experimental import pallas as pl
from jax.experimental.pallas import tpu as pltpu

_NUM_EMBEDDINGS = 1024
_EMBEDDING_DIM = 64
_BLOCK = 4096


def _vq_kernel(n_tokens, grid, x_ref, emb_ref, x2_ref, e2_ref, q_ref, idx_ref,
               loss_ref, perp_ref, counts_scr, loss_scr):
    i = pl.program_id(0)
    x = x_ref[...]
    emb = emb_ref[...]
    xt = x.T
    mm2_t = jnp.dot(emb + emb, xt,
                    preferred_element_type=jnp.float32)
    scores_t = (x2_ref[...] - mm2_t) + e2_ref[...]
    minval = jnp.min(scores_t, axis=0)
    iota_c = jax.lax.broadcasted_iota(jnp.int32, (_NUM_EMBEDDINGS, _BLOCK), 0)
    masked = jnp.where(scores_t == minval[None, :], iota_c,
                       _NUM_EMBEDDINGS)
    idx = jnp.min(masked, axis=0).astype(jnp.int32)
    onehot_t = (iota_c == idx[None, :]).astype(jnp.float32)
    q = jax.lax.dot_general(
        onehot_t, emb, (((0,), (0,)), ((), ())),
        preferred_element_type=jnp.float32)
    q_ref[...] = x + (q - x)
    idx_ref[0, 0, :] = idx

    diff = q - x
    part_loss = jnp.sum(diff * diff)
    ones_row = jnp.ones((1, _BLOCK), jnp.float32)
    part_counts = jax.lax.dot_general(
        ones_row, onehot_t, (((1,), (1,)), ((), ())),
        preferred_element_type=jnp.float32)

    @pl.when(i == 0)
    def _init():
        loss_scr[0, 0] = 0.0
        counts_scr[...] = jnp.zeros_like(counts_scr)

    loss_scr[0, 0] += part_loss
    counts_scr[...] += part_counts

    @pl.when(i == grid - 1)
    def _finalize():
        loss_ref[0, 0] = loss_scr[0, 0] / (n_tokens * _EMBEDDING_DIM)
        p = counts_scr[0, :] * (1.0 / n_tokens)
        perp_ref[0, 0] = jnp.exp(-jnp.sum(p * jnp.log(p + 1e-10)))


def kernel(inputs, embedding):
    input_shape = inputs.shape
    flat = inputs.reshape(-1, _EMBEDDING_DIM)
    n_tokens = flat.shape[0]
    grid = n_tokens // _BLOCK
    x2 = jnp.sum(flat ** 2, axis=1).reshape(1, n_tokens)
    e2 = jnp.sum(embedding ** 2, axis=1).reshape(_NUM_EMBEDDINGS, 1)

    quantized, idx3, loss, perp = pl.pallas_call(
        functools.partial(_vq_kernel, n_tokens, grid),
        grid=(grid,),
        in_specs=[
            pl.BlockSpec((_BLOCK, _EMBEDDING_DIM), lambda i: (i, 0)),
            pl.BlockSpec((_NUM_EMBEDDINGS, _EMBEDDING_DIM), lambda i: (0, 0)),
            pl.BlockSpec((1, _BLOCK), lambda i: (0, i)),
            pl.BlockSpec((_NUM_EMBEDDINGS, 1), lambda i: (0, 0)),
        ],
        out_specs=[
            pl.BlockSpec((_BLOCK, _EMBEDDING_DIM), lambda i: (i, 0)),
            pl.BlockSpec((1, 1, _BLOCK), lambda i: (i, 0, 0)),
            pl.BlockSpec(memory_space=pltpu.SMEM, block_shape=(1, 1),
                         index_map=lambda i: (0, 0)),
            pl.BlockSpec(memory_space=pltpu.SMEM, block_shape=(1, 1),
                         index_map=lambda i: (0, 0)),
        ],
        out_shape=[
            jax.ShapeDtypeStruct((n_tokens, _EMBEDDING_DIM), jnp.float32),
            jax.ShapeDtypeStruct((grid, 1, _BLOCK), jnp.int32),
            jax.ShapeDtypeStruct((1, 1), jnp.float32),
            jax.ShapeDtypeStruct((1, 1), jnp.float32),
        ],
        scratch_shapes=[
            pltpu.VMEM((1, _NUM_EMBEDDINGS), jnp.float32),
            pltpu.SMEM((1, 1), jnp.float32),
        ],
    )(flat, embedding, x2, e2)

    quantized = quantized.reshape(input_shape)
    indices = idx3.reshape(input_shape[:-1])
    return (quantized, loss.reshape(()), indices, perp.reshape(()))


# speedup vs baseline: 1.5607x; 1.0144x over previous
import functools

import jax
import jax.numpy as jnp
from jax.experimental import pallas as pl
from jax.experimental.pallas import tpu as pltpu

_NUM_EMBEDDINGS = 1024
_EMBEDDING_DIM = 64
_BLOCK = 2048


def _vq_kernel(n_tokens, grid, x_ref, emb_ref, x2_ref, e2_ref, q_ref, idx_ref,
               loss_ref, perp_ref, counts_scr, loss_scr):
    i = pl.program_id(0)
    x = x_ref[...]
    emb = emb_ref[...]
    xt = x.T
    mm2_t = jnp.dot(emb + emb, xt,
                    preferred_element_type=jnp.float32)
    scores_t = (x2_ref[...] - mm2_t) + e2_ref[...]
    minval = jnp.min(scores_t, axis=0)
    iota_c = jax.lax.broadcasted_iota(jnp.int32, (_NUM_EMBEDDINGS, _BLOCK), 0)
    masked = jnp.where(scores_t == minval[None, :], iota_c,
                       _NUM_EMBEDDINGS)
    idx = jnp.min(masked, axis=0).astype(jnp.int32)
    onehot_t = (iota_c == idx[None, :]).astype(jnp.float32)
    q = jax.lax.dot_general(
        onehot_t, emb, (((0,), (0,)), ((), ())),
        preferred_element_type=jnp.float32)
    q_ref[...] = x + (q - x)
    idx_ref[0, 0, :] = idx

    diff = q - x
    part_loss = jnp.sum(diff * diff)
    ones_row = jnp.ones((1, _BLOCK), jnp.float32)
    part_counts = jax.lax.dot_general(
        ones_row, onehot_t, (((1,), (1,)), ((), ())),
        preferred_element_type=jnp.float32)

    @pl.when(i == 0)
    def _init():
        loss_scr[0, 0] = 0.0
        counts_scr[...] = jnp.zeros_like(counts_scr)

    loss_scr[0, 0] += part_loss
    counts_scr[...] += part_counts

    @pl.when(i == grid - 1)
    def _finalize():
        loss_ref[0, 0] = loss_scr[0, 0] / (n_tokens * _EMBEDDING_DIM)
        p = counts_scr[0, :] * (1.0 / n_tokens)
        perp_ref[0, 0] = jnp.exp(-jnp.sum(p * jnp.log(p + 1e-10)))


def kernel(inputs, embedding):
    input_shape = inputs.shape
    flat = inputs.reshape(-1, _EMBEDDING_DIM)
    n_tokens = flat.shape[0]
    grid = n_tokens // _BLOCK
    x2 = jnp.sum(flat ** 2, axis=1).reshape(1, n_tokens)
    e2 = jnp.sum(embedding ** 2, axis=1).reshape(_NUM_EMBEDDINGS, 1)

    quantized, idx3, loss, perp = pl.pallas_call(
        functools.partial(_vq_kernel, n_tokens, grid),
        grid=(grid,),
        in_specs=[
            pl.BlockSpec((_BLOCK, _EMBEDDING_DIM), lambda i: (i, 0)),
            pl.BlockSpec((_NUM_EMBEDDINGS, _EMBEDDING_DIM), lambda i: (0, 0)),
            pl.BlockSpec((1, _BLOCK), lambda i: (0, i)),
            pl.BlockSpec((_NUM_EMBEDDINGS, 1), lambda i: (0, 0)),
        ],
        out_specs=[
            pl.BlockSpec((_BLOCK, _EMBEDDING_DIM), lambda i: (i, 0)),
            pl.BlockSpec((1, 1, _BLOCK), lambda i: (i, 0, 0)),
            pl.BlockSpec(memory_space=pltpu.SMEM, block_shape=(1, 1),
                         index_map=lambda i: (0, 0)),
            pl.BlockSpec(memory_space=pltpu.SMEM, block_shape=(1, 1),
                         index_map=lambda i: (0, 0)),
        ],
        out_shape=[
            jax.ShapeDtypeStruct((n_tokens, _EMBEDDING_DIM), jnp.float32),
            jax.ShapeDtypeStruct((grid, 1, _BLOCK), jnp.int32),
            jax.ShapeDtypeStruct((1, 1), jnp.float32),
            jax.ShapeDtypeStruct((1, 1), jnp.float32),
        ],
        scratch_shapes=[
            pltpu.VMEM((1, _NUM_EMBEDDINGS), jnp.float32),
            pltpu.SMEM((1, 1), jnp.float32),
        ],
    )(flat, embedding, x2, e2)

    quantized = quantized.reshape(input_shape)
    indices = idx3.reshape(input_shape[:-1])
    return (quantized, loss.reshape(()), indices, perp.reshape(()))
